# baseline (speedup 1.0000x reference)
import functools

import jax
import jax.numpy as jnp
from jax import lax
from jax.experimental import pallas as pl
from jax.experimental.pallas import tpu as pltpu
from jax.experimental.pallas import tpu_sc as plsc

B, S, H = 128, 512, 768
R = 32
NBLK_MAX = S // R
NG = H // 16
NEG = float(jnp.finfo(jnp.float32).min)


def _pool_body(t1_hbm, t2_hbm, lo_hbm, hi_hbm, inv_hbm, out_hbm,
               ids_v, inv_v, buf_v, out_v, sem):
    info = plsc.get_sparse_core_info()
    nc = info.num_cores
    ns = info.num_subcores
    nw = nc * ns
    wid = lax.axis_index("s") * nc + lax.axis_index("c")

    pltpu.sync_copy(lo_hbm, ids_v.at[0])
    pltpu.sync_copy(hi_hbm, ids_v.at[1])
    pltpu.sync_copy(inv_hbm, inv_v)

    def read_lane(ref, b):
        idx = jnp.zeros((16,), jnp.int32) + b
        return plsc.load_gather(ref, [idx])[0]

    def do_batch(j, carry):
        b = wid + nw * j
        lo = read_lane(ids_v.at[0], b)
        hi = read_lane(ids_v.at[1], b)
        blk_lo = lax.div(lo, R)
        blk_hi = lax.div(hi, R)
        nblk = blk_hi - blk_lo + 1

        def start_chunk(src, k):
            blk = blk_lo + k
            p = lax.rem(k, 2)
            pltpu.async_copy(src.at[b, pl.ds(blk * R, R), :],
                             buf_v.at[p], sem.at[p])

        def wait_chunk(src, k):
            blk = blk_lo + k
            p = lax.rem(k, 2)
            pltpu.make_async_copy(src.at[b, pl.ds(blk * R, R), :],
                                  buf_v.at[p], sem.at[p]).wait()

        def max_chunk(k, accs):
            @pl.when(k + 1 < nblk)
            def _():
                start_chunk(t1_hbm, k + 1)
            wait_chunk(t1_hbm, k)
            p = lax.rem(k, 2)
            base = (blk_lo + k) * R

            def row(r, accs):
                rc = jnp.clip(base + r, lo, hi) - base
                return tuple(
                    jnp.maximum(accs[c], buf_v[p, rc, pl.ds(16 * c, 16)])
                    for c in range(NG))
            return lax.fori_loop(0, R, row, accs)

        start_chunk(t1_hbm, 0)
        acc0 = tuple(jnp.full((16,), NEG, jnp.float32) for _ in range(NG))
        mx = lax.fori_loop(0, nblk, max_chunk, acc0)
        for c in range(NG):
            out_v[pl.ds(16 * c, 16)] = mx[c]

        def sum_chunk(k, accs):
            @pl.when(k + 1 < nblk)
            def _():
                start_chunk(t2_hbm, k + 1)
            wait_chunk(t2_hbm, k)
            p = lax.rem(k, 2)
            base = (blk_lo + k) * R

            def row(r, accs):
                pos = base + r
                w = jnp.where((pos >= lo) & (pos <= hi),
                              jnp.float32(1.0), jnp.float32(0.0))
                return tuple(
                    accs[c] + buf_v[p, r, pl.ds(16 * c, 16)] * w
                    for c in range(NG))
            return lax.fori_loop(0, R, row, accs)

        start_chunk(t2_hbm, 0)
        acc0 = tuple(jnp.zeros((16,), jnp.float32) for _ in range(NG))
        sm = lax.fori_loop(0, nblk, sum_chunk, acc0)
        inv = read_lane(inv_v, b)
        for c in range(NG):
            out_v[pl.ds(H + 16 * c, 16)] = sm[c] * inv

        pltpu.sync_copy(out_v, out_hbm.at[b])
        return carry

    lax.fori_loop(0, B // nw, do_batch, 0)


@jax.jit
def _pooling(tensor1, tensor2, start_ids, end_ids):
    lo = jnp.minimum(start_ids, end_ids)
    hi = jnp.maximum(start_ids, end_ids)
    inv = 1.0 / (hi - lo + 1).astype(jnp.float32)
    mesh = plsc.VectorSubcoreMesh(core_axis_name="c", subcore_axis_name="s")
    return pl.kernel(
        _pool_body,
        mesh=mesh,
        compiler_params=pltpu.CompilerParams(needs_layout_passes=False),
        out_type=jax.ShapeDtypeStruct((B, 2 * H), jnp.float32),
        scratch_types=[
            pltpu.VMEM((2, B), jnp.int32),
            pltpu.VMEM((B,), jnp.float32),
            pltpu.VMEM((2, R, H), jnp.float32),
            pltpu.VMEM((2 * H,), jnp.float32),
            pltpu.SemaphoreType.DMA((2,)),
        ],
    )(tensor1, tensor2, lo, hi, inv)


def kernel(tensor1, tensor2, start_ids, end_ids):
    return _pooling(tensor1, tensor2, start_ids, end_ids)


# speedup vs baseline: 1.0953x; 1.0953x over previous
import jax
import jax.numpy as jnp
from jax import lax
from jax.experimental import pallas as pl
from jax.experimental.pallas import tpu as pltpu
from jax.experimental.pallas import tpu_sc as plsc

B, S, H = 128, 512, 768
R = 16
NG = H // 16
NEG = float(jnp.finfo(jnp.float32).min)


def _pool_body(t1_hbm, t2_hbm, lo_hbm, hi_hbm, inv_hbm, out_hbm,
               ids_v, inv_v, buf_v, out_v, sem):
    info = plsc.get_sparse_core_info()
    nc = info.num_cores
    ns = info.num_subcores
    nw = nc * ns
    wid = lax.axis_index("s") * nc + lax.axis_index("c")

    pltpu.sync_copy(lo_hbm, ids_v.at[0])
    pltpu.sync_copy(hi_hbm, ids_v.at[1])
    pltpu.sync_copy(inv_hbm, inv_v)

    def read_lane(ref, b):
        idx = jnp.zeros((16,), jnp.int32) + b
        return plsc.load_gather(ref, [idx])[0]

    units = []
    wid2 = lax.rem(wid + nw // 2, nw)
    for k in range(B // nw):
        units.append((t1_hbm, wid + nw * k, True))
    for k in range(B // nw):
        units.append((t2_hbm, wid2 + nw * k, False))

    meta = []
    for src, b, is_max in units:
        lo = read_lane(ids_v.at[0], b)
        hi = read_lane(ids_v.at[1], b)
        blk_lo = lax.div(lo, R)
        nblk = lax.div(hi, R) - blk_lo + 1
        meta.append((src, b, is_max, lo, hi, blk_lo, nblk))

    def start_chunk(src, b, blk, par):
        pltpu.async_copy(src.at[b, pl.ds(blk * R, R), :],
                         buf_v.at[par], sem.at[par])

    def wait_chunk(src, b, blk, par):
        pltpu.make_async_copy(src.at[b, pl.ds(blk * R, R), :],
                              buf_v.at[par], sem.at[par]).wait()

    start_chunk(meta[0][0], meta[0][1], meta[0][5], 0)
    par0 = jnp.int32(0)

    for p, (src, b, is_max, lo, hi, blk_lo, nblk) in enumerate(meta):
        nxt = meta[p + 1] if p + 1 < len(meta) else None

        def chunk_body(k, accs, src=src, b=b, is_max=is_max, lo=lo, hi=hi,
                       blk_lo=blk_lo, nblk=nblk, nxt=nxt, par0=par0):
            par = lax.rem(par0 + k, 2)
            parn = lax.rem(par0 + k + 1, 2)

            @pl.when(k + 1 < nblk)
            def _():
                start_chunk(src, b, blk_lo + k + 1, parn)
            if nxt is not None:
                @pl.when(k + 1 == nblk)
                def _():
                    start_chunk(nxt[0], nxt[1], nxt[5], parn)
            wait_chunk(src, b, blk_lo + k, par)
            base = (blk_lo + k) * R

            if is_max:
                def row(r, accs):
                    rc = jnp.clip(base + r, lo, hi) - base
                    return tuple(
                        jnp.maximum(accs[c], buf_v[par, rc, pl.ds(16 * c, 16)])
                        for c in range(NG))
            else:
                def row(r, accs):
                    pos = base + r
                    w = jnp.where((pos >= lo) & (pos <= hi),
                                  jnp.float32(1.0), jnp.float32(0.0))
                    return tuple(
                        accs[c] + buf_v[par, r, pl.ds(16 * c, 16)] * w
                        for c in range(NG))
            return lax.fori_loop(0, R, row, accs)

        init = NEG if is_max else 0.0
        acc0 = tuple(jnp.full((16,), init, jnp.float32) for _ in range(NG))
        accs = lax.fori_loop(0, nblk, chunk_body, acc0)

        if is_max:
            for c in range(NG):
                out_v[pl.ds(16 * c, 16)] = accs[c]
            pltpu.sync_copy(out_v, out_hbm.at[b, pl.ds(0, H)])
        else:
            inv = read_lane(inv_v, b)
            for c in range(NG):
                out_v[pl.ds(16 * c, 16)] = accs[c] * inv
            pltpu.sync_copy(out_v, out_hbm.at[b, pl.ds(H, H)])
        par0 = lax.rem(par0 + nblk, 2)


@jax.jit
def _pooling(tensor1, tensor2, start_ids, end_ids):
    lo = jnp.minimum(start_ids, end_ids)
    hi = jnp.maximum(start_ids, end_ids)
    inv = 1.0 / (hi - lo + 1).astype(jnp.float32)
    mesh = plsc.VectorSubcoreMesh(core_axis_name="c", subcore_axis_name="s")
    return pl.kernel(
        _pool_body,
        mesh=mesh,
        compiler_params=pltpu.CompilerParams(needs_layout_passes=False),
        out_type=jax.ShapeDtypeStruct((B, 2 * H), jnp.float32),
        scratch_types=[
            pltpu.VMEM((2, B), jnp.int32),
            pltpu.VMEM((B,), jnp.float32),
            pltpu.VMEM((2, R, H), jnp.float32),
            pltpu.VMEM((H,), jnp.float32),
            pltpu.SemaphoreType.DMA((2,)),
        ],
    )(tensor1, tensor2, lo, hi, inv)


def kernel(tensor1, tensor2, start_ids, end_ids):
    return _pooling(tensor1, tensor2, start_ids, end_ids)


# speedup vs baseline: 1.0984x; 1.0029x over previous
import jax
import jax.numpy as jnp
from jax import lax
from jax.experimental import pallas as pl
from jax.experimental.pallas import tpu as pltpu
from jax.experimental.pallas import tpu_sc as plsc

B, S, H = 128, 512, 768
R = 16
NG = H // 16
NEG = float(jnp.finfo(jnp.float32).min)


def _pool_body(t1_hbm, t2_hbm, lo_hbm, hi_hbm, inv_hbm, out_hbm,
               ids_v, inv_v, buf_v, out_v, sem):
    info = plsc.get_sparse_core_info()
    nc = info.num_cores
    ns = info.num_subcores
    nw = nc * ns
    wid = lax.axis_index("s") * nc + lax.axis_index("c")

    pltpu.sync_copy(lo_hbm, ids_v.at[0])
    pltpu.sync_copy(hi_hbm, ids_v.at[1])
    pltpu.sync_copy(inv_hbm, inv_v)

    def read_lane(ref, b):
        idx = jnp.zeros((16,), jnp.int32) + b
        return plsc.load_gather(ref, [idx])[0]

    units = []
    wid2 = lax.rem(wid + nw // 2, nw)
    for k in range(B // nw):
        units.append((t1_hbm, wid + nw * k, True))
    for k in range(B // nw):
        units.append((t2_hbm, wid2 + nw * k, False))

    meta = []
    for src, b, is_max in units:
        lo = read_lane(ids_v.at[0], b)
        hi = read_lane(ids_v.at[1], b)
        blk_lo = lax.div(lo, R)
        nblk = lax.div(hi, R) - blk_lo + 1
        meta.append((src, b, is_max, lo, hi, blk_lo, nblk))

    def start_chunk(src, b, blk, par):
        pltpu.async_copy(src.at[b, pl.ds(blk * R, R), :],
                         buf_v.at[par], sem.at[par])

    def wait_chunk(src, b, blk, par):
        pltpu.make_async_copy(src.at[b, pl.ds(blk * R, R), :],
                              buf_v.at[par], sem.at[par]).wait()

    start_chunk(meta[0][0], meta[0][1], meta[0][5], 0)
    par0 = jnp.int32(0)

    for p, (src, b, is_max, lo, hi, blk_lo, nblk) in enumerate(meta):
        nxt = meta[p + 1] if p + 1 < len(meta) else None

        def chunk_body(k, accs, src=src, b=b, is_max=is_max, lo=lo, hi=hi,
                       blk_lo=blk_lo, nblk=nblk, nxt=nxt, par0=par0):
            par = lax.rem(par0 + k, 2)
            parn = lax.rem(par0 + k + 1, 2)

            @pl.when(k + 1 < nblk)
            def _():
                start_chunk(src, b, blk_lo + k + 1, parn)
            if nxt is not None:
                @pl.when(k + 1 == nblk)
                def _():
                    start_chunk(nxt[0], nxt[1], nxt[5], parn)
            wait_chunk(src, b, blk_lo + k, par)
            base = (blk_lo + k) * R

            GSUB = 16
            new_accs = list(accs)
            for g0 in range(0, NG, GSUB):
                if is_max:
                    def row(r, sub, g0=g0):
                        rc = jnp.clip(base + r, lo, hi) - base
                        return tuple(
                            jnp.maximum(sub[i],
                                        buf_v[par, rc, pl.ds(16 * (g0 + i), 16)])
                            for i in range(GSUB))
                else:
                    def row(r, sub, g0=g0):
                        pos = base + r
                        w = jnp.where((pos >= lo) & (pos <= hi),
                                      jnp.float32(1.0), jnp.float32(0.0))
                        return tuple(
                            sub[i] + buf_v[par, r, pl.ds(16 * (g0 + i), 16)] * w
                            for i in range(GSUB))
                sub = lax.fori_loop(0, R, row, tuple(accs[g0:g0 + GSUB]))
                new_accs[g0:g0 + GSUB] = list(sub)
            return tuple(new_accs)

        init = NEG if is_max else 0.0
        acc0 = tuple(jnp.full((16,), init, jnp.float32) for _ in range(NG))
        accs = lax.fori_loop(0, nblk, chunk_body, acc0)

        if is_max:
            for c in range(NG):
                out_v[pl.ds(16 * c, 16)] = accs[c]
            pltpu.sync_copy(out_v, out_hbm.at[b, pl.ds(0, H)])
        else:
            inv = read_lane(inv_v, b)
            for c in range(NG):
                out_v[pl.ds(16 * c, 16)] = accs[c] * inv
            pltpu.sync_copy(out_v, out_hbm.at[b, pl.ds(H, H)])
        par0 = lax.rem(par0 + nblk, 2)


@jax.jit
def _pooling(tensor1, tensor2, start_ids, end_ids):
    lo = jnp.minimum(start_ids, end_ids)
    hi = jnp.maximum(start_ids, end_ids)
    inv = 1.0 / (hi - lo + 1).astype(jnp.float32)
    mesh = plsc.VectorSubcoreMesh(core_axis_name="c", subcore_axis_name="s")
    return pl.kernel(
        _pool_body,
        mesh=mesh,
        compiler_params=pltpu.CompilerParams(needs_layout_passes=False),
        out_type=jax.ShapeDtypeStruct((B, 2 * H), jnp.float32),
        scratch_types=[
            pltpu.VMEM((2, B), jnp.int32),
            pltpu.VMEM((B,), jnp.float32),
            pltpu.VMEM((2, R, H), jnp.float32),
            pltpu.VMEM((H,), jnp.float32),
            pltpu.SemaphoreType.DMA((2,)),
        ],
    )(tensor1, tensor2, lo, hi, inv)


def kernel(tensor1, tensor2, start_ids, end_ids):
    return _pooling(tensor1, tensor2, start_ids, end_ids)


# speedup vs baseline: 1.2532x; 1.1409x over previous
import jax
import jax.numpy as jnp
from jax import lax
from jax.experimental import pallas as pl
from jax.experimental.pallas import tpu as pltpu
from jax.experimental.pallas import tpu_sc as plsc

B, S, H = 128, 512, 768
R = 32
NG = H // 16
NEG = float(jnp.finfo(jnp.float32).min)


def _pool_body(t1_hbm, t2_hbm, lo_hbm, hi_hbm, inv_hbm, out_hbm,
               ids_v, inv_v, buf_v, out_v, sem):
    info = plsc.get_sparse_core_info()
    nc = info.num_cores
    ns = info.num_subcores
    nw = nc * ns
    wid = lax.axis_index("s") * nc + lax.axis_index("c")

    pltpu.sync_copy(lo_hbm, ids_v.at[0])
    pltpu.sync_copy(hi_hbm, ids_v.at[1])
    pltpu.sync_copy(inv_hbm, inv_v)

    def read_lane(ref, b):
        idx = jnp.zeros((16,), jnp.int32) + b
        return plsc.load_gather(ref, [idx])[0]

    units = []
    wid2 = lax.rem(wid + nw // 2, nw)
    for k in range(B // nw):
        units.append((t1_hbm, wid + nw * k, True))
    for k in range(B // nw):
        units.append((t2_hbm, wid2 + nw * k, False))

    meta = []
    for src, b, is_max in units:
        lo = read_lane(ids_v.at[0], b)
        hi = read_lane(ids_v.at[1], b)
        blk_lo = lax.div(lo, R)
        nblk = lax.div(hi, R) - blk_lo + 1
        meta.append((src, b, is_max, lo, hi, blk_lo, nblk))

    def start_chunk(src, b, blk, par):
        pltpu.async_copy(src.at[b, pl.ds(blk * R, R), :],
                         buf_v.at[par], sem.at[par])

    def wait_chunk(src, b, blk, par):
        pltpu.make_async_copy(src.at[b, pl.ds(blk * R, R), :],
                              buf_v.at[par], sem.at[par]).wait()

    start_chunk(meta[0][0], meta[0][1], meta[0][5], 0)
    par0 = jnp.int32(0)

    for p, (src, b, is_max, lo, hi, blk_lo, nblk) in enumerate(meta):
        nxt = meta[p + 1] if p + 1 < len(meta) else None

        def chunk_body(k, accs, src=src, b=b, is_max=is_max, lo=lo, hi=hi,
                       blk_lo=blk_lo, nblk=nblk, nxt=nxt, par0=par0):
            par = lax.rem(par0 + k, 2)
            parn = lax.rem(par0 + k + 1, 2)

            @pl.when(k + 1 < nblk)
            def _():
                start_chunk(src, b, blk_lo + k + 1, parn)
            if nxt is not None:
                @pl.when(k + 1 == nblk)
                def _():
                    start_chunk(nxt[0], nxt[1], nxt[5], parn)
            wait_chunk(src, b, blk_lo + k, par)
            base = (blk_lo + k) * R

            GSUB = 16
            new_accs = list(accs)
            for g0 in range(0, NG, GSUB):
                if is_max:
                    def row(r, sub, g0=g0):
                        rc = jnp.clip(base + r, lo, hi) - base
                        return tuple(
                            jnp.maximum(sub[i],
                                        buf_v[par, rc, pl.ds(16 * (g0 + i), 16)])
                            for i in range(GSUB))
                else:
                    def row(r, sub, g0=g0):
                        pos = base + r
                        w = jnp.where((pos >= lo) & (pos <= hi),
                                      jnp.float32(1.0), jnp.float32(0.0))
                        return tuple(
                            sub[i] + buf_v[par, r, pl.ds(16 * (g0 + i), 16)] * w
                            for i in range(GSUB))
                sub = lax.fori_loop(0, R, row, tuple(accs[g0:g0 + GSUB]))
                new_accs[g0:g0 + GSUB] = list(sub)
            return tuple(new_accs)

        init = NEG if is_max else 0.0
        acc0 = tuple(jnp.full((16,), init, jnp.float32) for _ in range(NG))
        accs = lax.fori_loop(0, nblk, chunk_body, acc0)

        if is_max:
            for c in range(NG):
                out_v[pl.ds(16 * c, 16)] = accs[c]
            pltpu.sync_copy(out_v, out_hbm.at[b, pl.ds(0, H)])
        else:
            inv = read_lane(inv_v, b)
            for c in range(NG):
                out_v[pl.ds(16 * c, 16)] = accs[c] * inv
            pltpu.sync_copy(out_v, out_hbm.at[b, pl.ds(H, H)])
        par0 = lax.rem(par0 + nblk, 2)


@jax.jit
def _pooling(tensor1, tensor2, start_ids, end_ids):
    lo = jnp.minimum(start_ids, end_ids)
    hi = jnp.maximum(start_ids, end_ids)
    inv = 1.0 / (hi - lo + 1).astype(jnp.float32)
    mesh = plsc.VectorSubcoreMesh(core_axis_name="c", subcore_axis_name="s")
    return pl.kernel(
        _pool_body,
        mesh=mesh,
        compiler_params=pltpu.CompilerParams(needs_layout_passes=False),
        out_type=jax.ShapeDtypeStruct((B, 2 * H), jnp.float32),
        scratch_types=[
            pltpu.VMEM((2, B), jnp.int32),
            pltpu.VMEM((B,), jnp.float32),
            pltpu.VMEM((2, R, H), jnp.float32),
            pltpu.VMEM((H,), jnp.float32),
            pltpu.SemaphoreType.DMA((2,)),
        ],
    )(tensor1, tensor2, lo, hi, inv)


def kernel(tensor1, tensor2, start_ids, end_ids):
    return _pooling(tensor1, tensor2, start_ids, end_ids)


# speedup vs baseline: 1.2748x; 1.0173x over previous
import jax
import jax.numpy as jnp
from jax import lax
from jax.experimental import pallas as pl
from jax.experimental.pallas import tpu as pltpu
from jax.experimental.pallas import tpu_sc as plsc

B, S, H = 128, 512, 768
R = 64
NG = H // 16
NEG = float(jnp.finfo(jnp.float32).min)


def _pool_body(t1_hbm, t2_hbm, lo_hbm, hi_hbm, inv_hbm, out_hbm,
               ids_v, inv_v, buf_v, out_v, sem):
    info = plsc.get_sparse_core_info()
    nc = info.num_cores
    ns = info.num_subcores
    nw = nc * ns
    wid = lax.axis_index("s") * nc + lax.axis_index("c")

    pltpu.sync_copy(lo_hbm, ids_v.at[0])
    pltpu.sync_copy(hi_hbm, ids_v.at[1])
    pltpu.sync_copy(inv_hbm, inv_v)

    def read_lane(ref, b):
        idx = jnp.zeros((16,), jnp.int32) + b
        return plsc.load_gather(ref, [idx])[0]

    units = []
    wid2 = lax.rem(wid + nw // 2, nw)
    for k in range(B // nw):
        units.append((t1_hbm, wid + nw * k, True))
    for k in range(B // nw):
        units.append((t2_hbm, wid2 + nw * k, False))

    meta = []
    for src, b, is_max in units:
        lo = read_lane(ids_v.at[0], b)
        hi = read_lane(ids_v.at[1], b)
        blk_lo = lax.div(lo, R)
        nblk = lax.div(hi, R) - blk_lo + 1
        meta.append((src, b, is_max, lo, hi, blk_lo, nblk))

    def start_chunk(src, b, blk, par):
        pltpu.async_copy(src.at[b, pl.ds(blk * R, R), :],
                         buf_v.at[par], sem.at[par])

    def wait_chunk(src, b, blk, par):
        pltpu.make_async_copy(src.at[b, pl.ds(blk * R, R), :],
                              buf_v.at[par], sem.at[par]).wait()

    start_chunk(meta[0][0], meta[0][1], meta[0][5], 0)
    par0 = jnp.int32(0)

    for p, (src, b, is_max, lo, hi, blk_lo, nblk) in enumerate(meta):
        nxt = meta[p + 1] if p + 1 < len(meta) else None

        def chunk_body(k, accs, src=src, b=b, is_max=is_max, lo=lo, hi=hi,
                       blk_lo=blk_lo, nblk=nblk, nxt=nxt, par0=par0):
            par = lax.rem(par0 + k, 2)
            parn = lax.rem(par0 + k + 1, 2)

            @pl.when(k + 1 < nblk)
            def _():
                start_chunk(src, b, blk_lo + k + 1, parn)
            if nxt is not None:
                @pl.when(k + 1 == nblk)
                def _():
                    start_chunk(nxt[0], nxt[1], nxt[5], parn)
            wait_chunk(src, b, blk_lo + k, par)
            base = (blk_lo + k) * R

            GSUB = 16
            new_accs = list(accs)
            for g0 in range(0, NG, GSUB):
                if is_max:
                    def row(r, sub, g0=g0):
                        rc = jnp.clip(base + r, lo, hi) - base
                        return tuple(
                            jnp.maximum(sub[i],
                                        buf_v[par, rc, pl.ds(16 * (g0 + i), 16)])
                            for i in range(GSUB))
                else:
                    def row(r, sub, g0=g0):
                        pos = base + r
                        w = jnp.where((pos >= lo) & (pos <= hi),
                                      jnp.float32(1.0), jnp.float32(0.0))
                        return tuple(
                            sub[i] + buf_v[par, r, pl.ds(16 * (g0 + i), 16)] * w
                            for i in range(GSUB))
                sub = lax.fori_loop(0, R, row, tuple(accs[g0:g0 + GSUB]))
                new_accs[g0:g0 + GSUB] = list(sub)
            return tuple(new_accs)

        init = NEG if is_max else 0.0
        acc0 = tuple(jnp.full((16,), init, jnp.float32) for _ in range(NG))
        accs = lax.fori_loop(0, nblk, chunk_body, acc0)

        if is_max:
            for c in range(NG):
                out_v[pl.ds(16 * c, 16)] = accs[c]
            pltpu.sync_copy(out_v, out_hbm.at[b, pl.ds(0, H)])
        else:
            inv = read_lane(inv_v, b)
            for c in range(NG):
                out_v[pl.ds(16 * c, 16)] = accs[c] * inv
            pltpu.sync_copy(out_v, out_hbm.at[b, pl.ds(H, H)])
        par0 = lax.rem(par0 + nblk, 2)


@jax.jit
def _pooling(tensor1, tensor2, start_ids, end_ids):
    lo = jnp.minimum(start_ids, end_ids)
    hi = jnp.maximum(start_ids, end_ids)
    inv = 1.0 / (hi - lo + 1).astype(jnp.float32)
    mesh = plsc.VectorSubcoreMesh(core_axis_name="c", subcore_axis_name="s")
    return pl.kernel(
        _pool_body,
        mesh=mesh,
        compiler_params=pltpu.CompilerParams(needs_layout_passes=False),
        out_type=jax.ShapeDtypeStruct((B, 2 * H), jnp.float32),
        scratch_types=[
            pltpu.VMEM((2, B), jnp.int32),
            pltpu.VMEM((B,), jnp.float32),
            pltpu.VMEM((2, R, H), jnp.float32),
            pltpu.VMEM((H,), jnp.float32),
            pltpu.SemaphoreType.DMA((2,)),
        ],
    )(tensor1, tensor2, lo, hi, inv)


def kernel(tensor1, tensor2, start_ids, end_ids):
    return _pooling(tensor1, tensor2, start_ids, end_ids)
